# baseline (speedup 1.0000x reference)
import functools

import jax
import jax.numpy as jnp
from jax import lax
from jax.experimental import pallas as pl
from jax.experimental.pallas import tpu as pltpu
from jax.experimental.pallas import tpu_sc as plsc

N = 10000
D = 128
NC = 2
NS = 16
NW = NC * NS
CH = 128
SEC = 40
NSPLIT = 1
SUB = CH // NSPLIT
TILE_ROWS = 624
LAST_EXTRA = N - NS * TILE_ROWS


def _sc_agg_body(nch, x_hbm, src_hbm, dst_hbm, w_hbm, out_hbm,
                 acc_sh, sbuf, dbuf, wbuf,
                 r0, r1, gsem0, gsem1, ssem0, ssem1):
    cid = lax.axis_index("c")
    sid = lax.axis_index("s")
    wid = sid * NC + cid
    nsec = nch // SEC

    rows = [r0, r1]
    gsems = [gsem0, gsem1]
    ssems = [ssem0, ssem1]

    row0 = sid * TILE_ROWS

    def zero_r0(i, _):
        for j in range(D // 16):
            r0[i, pl.ds(j * 16, 16)] = jnp.zeros((16,), jnp.float32)
        return _
    lax.fori_loop(0, CH, zero_r0, None)

    for k in range(TILE_ROWS // CH):
        pltpu.sync_copy(r0, acc_sh.at[pl.ds(row0 + k * CH, CH)])
    rem = TILE_ROWS - (TILE_ROWS // CH) * CH
    pltpu.sync_copy(r0.at[pl.ds(0, rem)],
                    acc_sh.at[pl.ds(row0 + TILE_ROWS - rem, rem)])

    @pl.when(sid == NS - 1)
    def _():
        pltpu.sync_copy(r0.at[pl.ds(0, LAST_EXTRA)],
                        acc_sh.at[pl.ds(NS * TILE_ROWS, LAST_EXTRA)])

    def scale_chunk(rbuf, c):
        def grp(gi, _):
            w16 = wbuf[c, pl.ds(gi * 16, 16)]
            for i in range(16):
                e = gi * 16 + i
                w = w16[i]
                for j in range(D // 16):
                    rbuf[e, pl.ds(j * 16, 16)] = rbuf[e, pl.ds(j * 16, 16)] * w
            return _
        lax.fori_loop(0, CH // 16, grp, None)

    def stage_section(s):
        erow = wid * nch + s * SEC
        pltpu.sync_copy(src_hbm.at[pl.ds(erow, SEC)], sbuf)
        pltpu.sync_copy(dst_hbm.at[pl.ds(erow, SEC)], dbuf)
        pltpu.sync_copy(w_hbm.at[pl.ds(erow, SEC)], wbuf)

    def start_gather(c, p):
        for u in range(NSPLIT):
            pltpu.async_copy(x_hbm.at[sbuf.at[c, pl.ds(u * SUB, SUB)]],
                             rows[p].at[pl.ds(u * SUB, SUB)], gsems[p])

    def wait_gather(c, p):
        for u in range(NSPLIT):
            pltpu.make_async_copy(x_hbm.at[sbuf.at[c, pl.ds(u * SUB, SUB)]],
                                  rows[p].at[pl.ds(u * SUB, SUB)],
                                  gsems[p]).wait()

    def run_section():
        def pipe(hh, _):
            for p in range(2):
                q = 1 - p
                c = 2 * hh + p
                start_gather(c, p)
                wait_gather(c, p)

                @pl.when(c > 0)
                def _():
                    pltpu.make_async_copy(rows[q], acc_sh.at[dbuf.at[c - 1]],
                                          ssems[q]).wait()

                scale_chunk(rows[p], c)
                pltpu.async_copy(rows[p], acc_sh.at[dbuf.at[c]], ssems[p],
                                 add=True)
            return _
        lax.fori_loop(0, SEC // 2, pipe, None)
        pltpu.make_async_copy(rows[1], acc_sh.at[dbuf.at[SEC - 1]],
                              ssems[1]).wait()

    stage_section(0)
    plsc.subcore_barrier()
    run_section()
    for s in range(1, nsec):
        stage_section(s)
        run_section()

    plsc.subcore_barrier()

    pltpu.sync_copy(acc_sh.at[pl.ds(row0, TILE_ROWS)],
                    out_hbm.at[pl.ds(cid * N + row0, TILE_ROWS)])

    @pl.when(sid == NS - 1)
    def _():
        pltpu.sync_copy(
            acc_sh.at[pl.ds(NS * TILE_ROWS, LAST_EXTRA)],
            out_hbm.at[pl.ds(cid * N + NS * TILE_ROWS, LAST_EXTRA)])


def _sc_agg(x, src2, dst2, w2, nch):
    mesh = plsc.VectorSubcoreMesh(core_axis_name="c", subcore_axis_name="s")
    f = pl.kernel(
        functools.partial(_sc_agg_body, nch),
        out_type=jax.ShapeDtypeStruct((NC * N, D), jnp.float32),
        mesh=mesh,
        scratch_types=[
            pltpu.VMEM_SHARED((N, D), jnp.float32),
            pltpu.VMEM((SEC, CH), jnp.int32),
            pltpu.VMEM((SEC, CH), jnp.int32),
            pltpu.VMEM((SEC, CH), jnp.float32),
            pltpu.VMEM((CH, D), jnp.float32),
            pltpu.VMEM((CH, D), jnp.float32),
            pltpu.SemaphoreType.DMA,
            pltpu.SemaphoreType.DMA,
            pltpu.SemaphoreType.DMA,
            pltpu.SemaphoreType.DMA,
        ],
    )
    return f(x, src2, dst2, w2)


def _tc_finish_body(p0_ref, p1_ref, w_ref, b_ref, o_ref):
    wn = jax.nn.softplus(w_ref[...])
    agg = p0_ref[...] + p1_ref[...]
    h = jnp.dot(agg, wn, preferred_element_type=jnp.float32)
    o_ref[...] = jnp.maximum(h + b_ref[...], 0.0)


def _tc_finish(partials, W, bias):
    nb = 10
    blk = N // nb
    return pl.pallas_call(
        _tc_finish_body,
        grid=(nb,),
        in_specs=[
            pl.BlockSpec((blk, D), lambda i: (i, 0)),
            pl.BlockSpec((blk, D), lambda i: (i + nb, 0)),
            pl.BlockSpec((D, D), lambda i: (0, 0)),
            pl.BlockSpec((1, D), lambda i: (0, 0)),
        ],
        out_specs=pl.BlockSpec((blk, D), lambda i: (i, 0)),
        out_shape=jax.ShapeDtypeStruct((N, D), jnp.float32),
    )(partials, partials, W, bias.reshape(1, D))


def kernel(x, edge_index, edge_weight, W, bias):
    e = edge_weight.shape[0]
    grain = NW * CH * SEC
    e_pad = ((e + grain - 1) // grain) * grain
    nch = e_pad // (NW * CH)
    pad = e_pad - e
    src2 = jnp.pad(edge_index[0], (0, pad)).reshape(e_pad // CH, CH)
    dst2 = jnp.pad(edge_index[1], (0, pad)).reshape(e_pad // CH, CH)
    w2 = jnp.pad(edge_weight, (0, pad)).reshape(e_pad // CH, CH)
    partials = _sc_agg(x, src2, dst2, w2, nch)
    return _tc_finish(partials, W, bias)


# speedup vs baseline: 1.0806x; 1.0806x over previous
import functools

import jax
import jax.numpy as jnp
from jax import lax
from jax.experimental import pallas as pl
from jax.experimental.pallas import tpu as pltpu
from jax.experimental.pallas import tpu_sc as plsc

N = 10000
D = 128
NC = 2
NS = 16
NW = NC * NS
CH = 128
SEC = 40
NSPLIT = 4
SUB = CH // NSPLIT
TILE_ROWS = 624
LAST_EXTRA = N - NS * TILE_ROWS


def _sc_agg_body(nch, x_hbm, src_hbm, dst_hbm, w_hbm, out_hbm,
                 acc_sh, sbuf, dbuf, wbuf,
                 r0, r1, gsem0, gsem1, ssem0, ssem1):
    cid = lax.axis_index("c")
    sid = lax.axis_index("s")
    wid = sid * NC + cid
    nsec = nch // SEC

    rows = [r0, r1]
    gsems = [gsem0, gsem1]
    ssems = [ssem0, ssem1]

    row0 = sid * TILE_ROWS

    def zero_r0(i, _):
        for j in range(D // 16):
            r0[i, pl.ds(j * 16, 16)] = jnp.zeros((16,), jnp.float32)
        return _
    lax.fori_loop(0, CH, zero_r0, None)

    for k in range(TILE_ROWS // CH):
        pltpu.sync_copy(r0, acc_sh.at[pl.ds(row0 + k * CH, CH)])
    rem = TILE_ROWS - (TILE_ROWS // CH) * CH
    pltpu.sync_copy(r0.at[pl.ds(0, rem)],
                    acc_sh.at[pl.ds(row0 + TILE_ROWS - rem, rem)])

    @pl.when(sid == NS - 1)
    def _():
        pltpu.sync_copy(r0.at[pl.ds(0, LAST_EXTRA)],
                        acc_sh.at[pl.ds(NS * TILE_ROWS, LAST_EXTRA)])

    def scale_chunk(rbuf, c):
        def grp(gi, _):
            w16 = wbuf[c, pl.ds(gi * 16, 16)]
            for i in range(16):
                e = gi * 16 + i
                w = w16[i]
                for j in range(D // 16):
                    rbuf[e, pl.ds(j * 16, 16)] = rbuf[e, pl.ds(j * 16, 16)] * w
            return _
        lax.fori_loop(0, CH // 16, grp, None)

    def stage_section(s):
        erow = wid * nch + s * SEC
        pltpu.sync_copy(src_hbm.at[pl.ds(erow, SEC)], sbuf)
        pltpu.sync_copy(dst_hbm.at[pl.ds(erow, SEC)], dbuf)
        pltpu.sync_copy(w_hbm.at[pl.ds(erow, SEC)], wbuf)

    def start_gather(c, p):
        for u in range(NSPLIT):
            pltpu.async_copy(x_hbm.at[sbuf.at[c, pl.ds(u * SUB, SUB)]],
                             rows[p].at[pl.ds(u * SUB, SUB)], gsems[p])

    def wait_gather(c, p):
        for u in range(NSPLIT):
            pltpu.make_async_copy(x_hbm.at[sbuf.at[c, pl.ds(u * SUB, SUB)]],
                                  rows[p].at[pl.ds(u * SUB, SUB)],
                                  gsems[p]).wait()

    def run_section():
        start_gather(0, 0)

        def pipe(hh, _):
            for p in range(2):
                q = 1 - p
                c = 2 * hh + p
                wait_gather(c, p)

                @pl.when(c > 0)
                def _():
                    pltpu.make_async_copy(rows[q], acc_sh.at[dbuf.at[c - 1]],
                                          ssems[q]).wait()

                @pl.when(c < SEC - 1)
                def _():
                    start_gather(c + 1, q)

                scale_chunk(rows[p], c)
                pltpu.async_copy(rows[p], acc_sh.at[dbuf.at[c]], ssems[p],
                                 add=True)
            return _
        lax.fori_loop(0, SEC // 2, pipe, None)
        pltpu.make_async_copy(rows[1], acc_sh.at[dbuf.at[SEC - 1]],
                              ssems[1]).wait()

    stage_section(0)
    plsc.subcore_barrier()
    run_section()
    for s in range(1, nsec):
        stage_section(s)
        run_section()

    plsc.subcore_barrier()

    pltpu.sync_copy(acc_sh.at[pl.ds(row0, TILE_ROWS)],
                    out_hbm.at[pl.ds(cid * N + row0, TILE_ROWS)])

    @pl.when(sid == NS - 1)
    def _():
        pltpu.sync_copy(
            acc_sh.at[pl.ds(NS * TILE_ROWS, LAST_EXTRA)],
            out_hbm.at[pl.ds(cid * N + NS * TILE_ROWS, LAST_EXTRA)])


def _sc_agg(x, src2, dst2, w2, nch):
    mesh = plsc.VectorSubcoreMesh(core_axis_name="c", subcore_axis_name="s")
    f = pl.kernel(
        functools.partial(_sc_agg_body, nch),
        out_type=jax.ShapeDtypeStruct((NC * N, D), jnp.float32),
        mesh=mesh,
        scratch_types=[
            pltpu.VMEM_SHARED((N, D), jnp.float32),
            pltpu.VMEM((SEC, CH), jnp.int32),
            pltpu.VMEM((SEC, CH), jnp.int32),
            pltpu.VMEM((SEC, CH), jnp.float32),
            pltpu.VMEM((CH, D), jnp.float32),
            pltpu.VMEM((CH, D), jnp.float32),
            pltpu.SemaphoreType.DMA,
            pltpu.SemaphoreType.DMA,
            pltpu.SemaphoreType.DMA,
            pltpu.SemaphoreType.DMA,
        ],
    )
    return f(x, src2, dst2, w2)


def _tc_finish_body(p0_ref, p1_ref, w_ref, b_ref, o_ref):
    wn = jax.nn.softplus(w_ref[...])
    agg = p0_ref[...] + p1_ref[...]
    h = jnp.dot(agg, wn, preferred_element_type=jnp.float32)
    o_ref[...] = jnp.maximum(h + b_ref[...], 0.0)


def _tc_finish(partials, W, bias):
    nb = 10
    blk = N // nb
    return pl.pallas_call(
        _tc_finish_body,
        grid=(nb,),
        in_specs=[
            pl.BlockSpec((blk, D), lambda i: (i, 0)),
            pl.BlockSpec((blk, D), lambda i: (i + nb, 0)),
            pl.BlockSpec((D, D), lambda i: (0, 0)),
            pl.BlockSpec((1, D), lambda i: (0, 0)),
        ],
        out_specs=pl.BlockSpec((blk, D), lambda i: (i, 0)),
        out_shape=jax.ShapeDtypeStruct((N, D), jnp.float32),
    )(partials, partials, W, bias.reshape(1, D))


def kernel(x, edge_index, edge_weight, W, bias):
    e = edge_weight.shape[0]
    grain = NW * CH * SEC
    e_pad = ((e + grain - 1) // grain) * grain
    nch = e_pad // (NW * CH)
    pad = e_pad - e
    src2 = jnp.pad(edge_index[0], (0, pad)).reshape(e_pad // CH, CH)
    dst2 = jnp.pad(edge_index[1], (0, pad)).reshape(e_pad // CH, CH)
    w2 = jnp.pad(edge_weight, (0, pad)).reshape(e_pad // CH, CH)
    partials = _sc_agg(x, src2, dst2, w2, nch)
    return _tc_finish(partials, W, bias)


# speedup vs baseline: 1.0833x; 1.0025x over previous
import functools

import jax
import jax.numpy as jnp
from jax import lax
from jax.experimental import pallas as pl
from jax.experimental.pallas import tpu as pltpu
from jax.experimental.pallas import tpu_sc as plsc

N = 10000
D = 128
NC = 2
NS = 16
NW = NC * NS
CH = 128
SEC = 40
NSPLIT = 4
SUB = CH // NSPLIT
TILE_ROWS = 624
LAST_EXTRA = N - NS * TILE_ROWS


def _sc_agg_body(nch, x_hbm, src_hbm, dst_hbm, w_hbm, out_hbm,
                 acc_sh, sbuf, dbuf, wbuf,
                 r0, r1, gsem0, gsem1, ssem0, ssem1):
    cid = lax.axis_index("c")
    sid = lax.axis_index("s")
    wid = sid * NC + cid
    nsec = nch // SEC

    rows = [r0, r1]
    gsems = [gsem0, gsem1]
    ssems = [ssem0, ssem1]

    row0 = sid * TILE_ROWS

    def zero_r0(i, _):
        for j in range(D // 16):
            r0[i, pl.ds(j * 16, 16)] = jnp.zeros((16,), jnp.float32)
        return _
    lax.fori_loop(0, CH, zero_r0, None)

    for k in range(TILE_ROWS // CH):
        pltpu.sync_copy(r0, acc_sh.at[pl.ds(row0 + k * CH, CH)])
    rem = TILE_ROWS - (TILE_ROWS // CH) * CH
    pltpu.sync_copy(r0.at[pl.ds(0, rem)],
                    acc_sh.at[pl.ds(row0 + TILE_ROWS - rem, rem)])

    @pl.when(sid == NS - 1)
    def _():
        pltpu.sync_copy(r0.at[pl.ds(0, LAST_EXTRA)],
                        acc_sh.at[pl.ds(NS * TILE_ROWS, LAST_EXTRA)])

    def scale_chunk(rbuf, c):
        def grp(gi, _):
            w16 = wbuf[c, pl.ds(gi * 16, 16)]
            for i in range(16):
                e = gi * 16 + i
                w = w16[i]
                for j in range(D // 16):
                    rbuf[e, pl.ds(j * 16, 16)] = rbuf[e, pl.ds(j * 16, 16)] * w
            return _
        lax.fori_loop(0, CH // 16, grp, None)

    def stage_section(s):
        erow = wid * nch + s * SEC
        pltpu.sync_copy(src_hbm.at[pl.ds(erow, SEC)], sbuf)
        pltpu.sync_copy(dst_hbm.at[pl.ds(erow, SEC)], dbuf)
        pltpu.sync_copy(w_hbm.at[pl.ds(erow, SEC)], wbuf)

    def start_gather(c, p):
        for u in range(NSPLIT):
            pltpu.async_copy(x_hbm.at[sbuf.at[c, pl.ds(u * SUB, SUB)]],
                             rows[p].at[pl.ds(u * SUB, SUB)], gsems[p])

    def wait_gather(c, p):
        for u in range(NSPLIT):
            pltpu.make_async_copy(x_hbm.at[sbuf.at[c, pl.ds(u * SUB, SUB)]],
                                  rows[p].at[pl.ds(u * SUB, SUB)],
                                  gsems[p]).wait()

    def run_section():
        start_gather(0, 0)

        def pipe(hh, _):
            for p in range(2):
                q = 1 - p
                c = 2 * hh + p
                wait_gather(c, p)

                @pl.when(c < SEC - 1)
                def _():
                    start_gather(c + 1, q)

                scale_chunk(rows[p], c)
                pltpu.sync_copy(rows[p], acc_sh.at[dbuf.at[c]], add=True)
            return _
        lax.fori_loop(0, SEC // 2, pipe, None)

    stage_section(0)
    plsc.subcore_barrier()
    run_section()
    for s in range(1, nsec):
        stage_section(s)
        run_section()

    plsc.subcore_barrier()

    pltpu.sync_copy(acc_sh.at[pl.ds(row0, TILE_ROWS)],
                    out_hbm.at[pl.ds(cid * N + row0, TILE_ROWS)])

    @pl.when(sid == NS - 1)
    def _():
        pltpu.sync_copy(
            acc_sh.at[pl.ds(NS * TILE_ROWS, LAST_EXTRA)],
            out_hbm.at[pl.ds(cid * N + NS * TILE_ROWS, LAST_EXTRA)])


def _sc_agg(x, src2, dst2, w2, nch):
    mesh = plsc.VectorSubcoreMesh(core_axis_name="c", subcore_axis_name="s")
    f = pl.kernel(
        functools.partial(_sc_agg_body, nch),
        out_type=jax.ShapeDtypeStruct((NC * N, D), jnp.float32),
        mesh=mesh,
        scratch_types=[
            pltpu.VMEM_SHARED((N, D), jnp.float32),
            pltpu.VMEM((SEC, CH), jnp.int32),
            pltpu.VMEM((SEC, CH), jnp.int32),
            pltpu.VMEM((SEC, CH), jnp.float32),
            pltpu.VMEM((CH, D), jnp.float32),
            pltpu.VMEM((CH, D), jnp.float32),
            pltpu.SemaphoreType.DMA,
            pltpu.SemaphoreType.DMA,
            pltpu.SemaphoreType.DMA,
            pltpu.SemaphoreType.DMA,
        ],
    )
    return f(x, src2, dst2, w2)


def _tc_finish_body(p0_ref, p1_ref, w_ref, b_ref, o_ref):
    wn = jax.nn.softplus(w_ref[...])
    agg = p0_ref[...] + p1_ref[...]
    h = jnp.dot(agg, wn, preferred_element_type=jnp.float32)
    o_ref[...] = jnp.maximum(h + b_ref[...], 0.0)


def _tc_finish(partials, W, bias):
    nb = 10
    blk = N // nb
    return pl.pallas_call(
        _tc_finish_body,
        grid=(nb,),
        in_specs=[
            pl.BlockSpec((blk, D), lambda i: (i, 0)),
            pl.BlockSpec((blk, D), lambda i: (i + nb, 0)),
            pl.BlockSpec((D, D), lambda i: (0, 0)),
            pl.BlockSpec((1, D), lambda i: (0, 0)),
        ],
        out_specs=pl.BlockSpec((blk, D), lambda i: (i, 0)),
        out_shape=jax.ShapeDtypeStruct((N, D), jnp.float32),
    )(partials, partials, W, bias.reshape(1, D))


def kernel(x, edge_index, edge_weight, W, bias):
    e = edge_weight.shape[0]
    grain = NW * CH * SEC
    e_pad = ((e + grain - 1) // grain) * grain
    nch = e_pad // (NW * CH)
    pad = e_pad - e
    src2 = jnp.pad(edge_index[0], (0, pad)).reshape(e_pad // CH, CH)
    dst2 = jnp.pad(edge_index[1], (0, pad)).reshape(e_pad // CH, CH)
    w2 = jnp.pad(edge_weight, (0, pad)).reshape(e_pad // CH, CH)
    partials = _sc_agg(x, src2, dst2, w2, nch)
    return _tc_finish(partials, W, bias)


# speedup vs baseline: 1.1284x; 1.0417x over previous
import functools

import jax
import jax.numpy as jnp
from jax import lax
from jax.experimental import pallas as pl
from jax.experimental.pallas import tpu as pltpu
from jax.experimental.pallas import tpu_sc as plsc

N = 10000
D = 128
NC = 2
NS = 16
NW = NC * NS
CH = 128
TILE_ROWS = 624
CR = 16


def _sc_agg_body(nch, x_hbm, src_hbm, dst_hbm, w_hbm, out_hbm,
                 acc_sh, zbuf, sidx, didx, wbuf, rows, sem):
    cid = lax.axis_index("c")
    sid = lax.axis_index("s")
    wid = sid * NC + cid
    epw = nch * CH

    row0 = sid * TILE_ROWS
    ncop = jnp.where(sid >= NS - 1, (N - (NS - 1) * TILE_ROWS) // CR,
                     TILE_ROWS // CR)

    def zero_zbuf(i, _):
        for j in range(D // 16):
            zbuf[i, pl.ds(j * 16, 16)] = jnp.zeros((16,), jnp.float32)
        return _
    lax.fori_loop(0, CR, zero_zbuf, None)

    def zero_acc(k, _):
        pltpu.sync_copy(zbuf, acc_sh.at[pl.ds(row0 + k * CR, CR)])
        return _
    lax.fori_loop(0, ncop, zero_acc, None)

    plsc.subcore_barrier()

    def chunk(c, _):
        base = wid * epw + c * CH
        pltpu.sync_copy(src_hbm.at[pl.ds(base, CH)], sidx)
        pltpu.sync_copy(dst_hbm.at[pl.ds(base, CH)], didx)
        pltpu.sync_copy(w_hbm.at[pl.ds(base, CH)], wbuf)
        pltpu.async_copy(x_hbm.at[sidx], rows, sem).wait()

        def scale_group(g, _):
            w16 = wbuf[pl.ds(g * 16, 16)]
            for i in range(16):
                e = g * 16 + i
                w = w16[i]
                for j in range(D // 16):
                    rows[e, pl.ds(j * 16, 16)] = rows[e, pl.ds(j * 16, 16)] * w
            return _
        lax.fori_loop(0, CH // 16, scale_group, None)

        pltpu.sync_copy(rows, acc_sh.at[didx], add=True)
        return _
    lax.fori_loop(0, nch, chunk, None)

    plsc.subcore_barrier()

    def writeback(k, _):
        r = row0 + k * CR
        pltpu.sync_copy(acc_sh.at[pl.ds(r, CR)], out_hbm.at[pl.ds(cid * N + r, CR)])
        return _
    lax.fori_loop(0, ncop, writeback, None)


def _sc_agg(x, src, dst, w, nch):
    mesh = plsc.VectorSubcoreMesh(core_axis_name="c", subcore_axis_name="s")
    f = pl.kernel(
        functools.partial(_sc_agg_body, nch),
        out_type=jax.ShapeDtypeStruct((NC * N, D), jnp.float32),
        mesh=mesh,
        scratch_types=[
            pltpu.VMEM_SHARED((N, D), jnp.float32),
            pltpu.VMEM((CR, D), jnp.float32),
            pltpu.VMEM((CH,), jnp.int32),
            pltpu.VMEM((CH,), jnp.int32),
            pltpu.VMEM((CH,), jnp.float32),
            pltpu.VMEM((CH, D), jnp.float32),
            pltpu.SemaphoreType.DMA,
        ],
    )
    return f(x, src, dst, w)


def _tc_finish_body(p0_ref, p1_ref, w_ref, b_ref, o_ref):
    wn = jax.nn.softplus(w_ref[...])
    agg = p0_ref[...] + p1_ref[...]
    h = jnp.dot(agg, wn, preferred_element_type=jnp.float32)
    o_ref[...] = jnp.maximum(h + b_ref[...], 0.0)


def _tc_finish(partials, W, bias):
    nb = 10
    blk = N // nb
    return pl.pallas_call(
        _tc_finish_body,
        grid=(nb,),
        in_specs=[
            pl.BlockSpec((blk, D), lambda i: (i, 0)),
            pl.BlockSpec((blk, D), lambda i: (i + nb, 0)),
            pl.BlockSpec((D, D), lambda i: (0, 0)),
            pl.BlockSpec((1, D), lambda i: (0, 0)),
        ],
        out_specs=pl.BlockSpec((blk, D), lambda i: (i, 0)),
        out_shape=jax.ShapeDtypeStruct((N, D), jnp.float32),
    )(partials, partials, W, bias.reshape(1, D))


def kernel(x, edge_index, edge_weight, W, bias):
    e = edge_weight.shape[0]
    grain = NW * CH
    e_pad = ((e + grain - 1) // grain) * grain
    nch = e_pad // (NW * CH)
    pad = e_pad - e
    src = jnp.pad(edge_index[0], (0, pad))
    dst = jnp.pad(edge_index[1], (0, pad))
    w = jnp.pad(edge_weight, (0, pad))
    partials = _sc_agg(x, src, dst, w, nch)
    return _tc_finish(partials, W, bias)
